# Initial kernel scaffold; baseline (speedup 1.0000x reference)
import functools

import jax
import jax.numpy as jnp
from jax import lax
from jax.experimental import pallas as pl
from jax.experimental.pallas import tpu as pltpu
from jax.experimental.pallas import tpu_sc as plsc

_N = 50000
_E = 800000
_NR = 3
_D = 64
_H = 64
_NG = 128
_LANES = 16

_NPAD = 50176
_SL = _NPAD // 16

_CH = 2560
_KROWS = _CH // 128
_NSUB = 10
_PERW = _CH * _NSUB
_NW = 32
_EPAD = _PERW * _NW

_NBLK = 8
_NB = _NPAD // _NBLK


def _wid(c, s):
    return c * 16 + s


def _fill_zero(ref, nwords):
    def body(i, _):
        ref[pl.ds(i * _LANES, _LANES)] = jnp.zeros((_LANES,), jnp.float32)
        return 0
    lax.fori_loop(0, nwords // _LANES, body, 0)


def _sc_a_body(dst_hbm, ty_hbm, out_hbm, dstb, tyb, idxb, valb, zb, spm, sem):
    c = lax.axis_index("c")
    s = lax.axis_index("s")
    wid = _wid(c, s)

    _fill_zero(zb, _SL)
    for k in range(3):
        pltpu.sync_copy(zb, spm.at[pl.ds(s * 3 * _SL + k * _SL, _SL)])
    plsc.subcore_barrier()

    ebase0 = pl.multiple_of(wid * _PERW, 128)

    def chunk(j, _):
        ebase = pl.multiple_of(ebase0 + j * _CH, 128)
        pltpu.sync_copy(dst_hbm.at[pl.ds(ebase, _CH)], dstb)
        pltpu.sync_copy(ty_hbm.at[pl.ds(ebase, _CH)], tyb)

        def vec(i, _):
            off = i * _LANES
            d = dstb[pl.ds(off, _LANES)]
            t = tyb[pl.ds(off, _LANES)]
            idxb[pl.ds(off, _LANES)] = t * _NPAD + d
            gi = ebase + off + lax.iota(jnp.int32, _LANES)
            valb[pl.ds(off, _LANES)] = jnp.where(gi < _E, 1.0, 0.0)
            return 0

        lax.fori_loop(0, _CH // _LANES, vec, 0)
        copies = []
        for k in range(_KROWS):
            copies.append(pltpu.make_async_copy(
                valb2.at[k], spm.at[idxb2.at[k]], sem))
        for cp in copies:
            cp.start()
        for cp in copies:
            cp.wait()
        return 0

    idxb2 = idxb.reshape(_KROWS, 128)
    valb2 = valb.reshape(_KROWS, 128)
    lax.fori_loop(0, _NSUB, chunk, 0)

    plsc.subcore_barrier()
    pltpu.sync_copy(spm.at[pl.ds(s * 3 * _SL, 3 * _SL)],
                    out_hbm.at[pl.ds(c * 3 * _NPAD + s * 3 * _SL, 3 * _SL)])


def _sc_b_body(src_hbm, dst_hbm, ty_hbm, cnt3_hbm, out_hbm,
               srcb, dstb, tyb, idxb, valb, pa, pb, stb, zb, stateb,
               spm_state, spm_hist, sem):
    c = lax.axis_index("c")
    s = lax.axis_index("s")
    wid = _wid(c, s)

    _fill_zero(zb, _SL)
    for k in range(24):
        pltpu.sync_copy(zb, spm_hist.at[pl.ds(s * 24 * _SL + k * _SL, _SL)])

    for r in range(3):
        pltpu.sync_copy(cnt3_hbm.at[pl.ds(r * _NPAD + s * _SL, _SL)], pa)
        pltpu.sync_copy(cnt3_hbm.at[pl.ds(3 * _NPAD + r * _NPAD + s * _SL, _SL)], pb)

        def vec(i, _):
            off = i * _LANES
            deg = pa[pl.ds(off, _LANES)] + pb[pl.ds(off, _LANES)]
            bit = jnp.where(deg > 0.0, jnp.int32(1 << r), jnp.int32(0))
            if r == 0:
                stb[pl.ds(off, _LANES)] = bit
            else:
                stb[pl.ds(off, _LANES)] = stb[pl.ds(off, _LANES)] + bit
            return 0

        lax.fori_loop(0, _SL // _LANES, vec, 0)
    pltpu.sync_copy(stb, spm_state.at[pl.ds(s * _SL, _SL)])
    plsc.subcore_barrier()

    pltpu.sync_copy(spm_state, stateb)

    ebase0 = pl.multiple_of(wid * _PERW, 128)
    idxb2 = idxb.reshape(_KROWS, 128)
    valb2 = valb.reshape(_KROWS, 128)

    def chunk(j, _):
        ebase = pl.multiple_of(ebase0 + j * _CH, 128)
        pltpu.sync_copy(src_hbm.at[pl.ds(ebase, _CH)], srcb)
        pltpu.sync_copy(dst_hbm.at[pl.ds(ebase, _CH)], dstb)
        pltpu.sync_copy(ty_hbm.at[pl.ds(ebase, _CH)], tyb)

        def vec(i, _):
            off = i * _LANES
            sv = srcb[pl.ds(off, _LANES)]
            st = plsc.load_gather(stateb, [sv])
            t = tyb[pl.ds(off, _LANES)]
            d = dstb[pl.ds(off, _LANES)]
            idxb[pl.ds(off, _LANES)] = (t * 8 + st) * _NPAD + d
            gi = ebase + off + lax.iota(jnp.int32, _LANES)
            valb[pl.ds(off, _LANES)] = jnp.where(gi < _E, 1.0, 0.0)
            return 0

        lax.fori_loop(0, _CH // _LANES, vec, 0)
        copies = []
        for k in range(_KROWS):
            copies.append(pltpu.make_async_copy(
                valb2.at[k], spm_hist.at[idxb2.at[k]], sem))
        for cp in copies:
            cp.start()
        for cp in copies:
            cp.wait()
        return 0

    lax.fori_loop(0, _NSUB, chunk, 0)

    plsc.subcore_barrier()
    pltpu.sync_copy(spm_hist.at[pl.ds(s * 24 * _SL, 24 * _SL)],
                    out_hbm.at[pl.ds(c * 24 * _NPAD + s * 24 * _SL, 24 * _SL)])


def _dotT(a, b):
    return lax.dot_general(a, b, (((0,), (0,)), ((), ())),
                           precision=lax.Precision.HIGHEST,
                           preferred_element_type=jnp.float32)


def _dot(a, b):
    return lax.dot_general(a, b, (((1,), (0,)), ((), ())),
                           precision=lax.Precision.HIGHEST,
                           preferred_element_type=jnp.float32)


def _tc_body(hp0, hp1, batch3, emb, w1, root1, bias1, w2, root2, bias2,
             lwp, lbp, out_ref, tscr, zscr, accg, accc):
    i = pl.program_id(0)

    @pl.when(i == 0)
    def _prologue():
        v = emb[...]
        base1 = _dot(v, root1[...]) + bias1[...]
        m3 = jnp.concatenate([_dot(v, w1[r]) for r in range(_NR)], axis=0)
        srow = lax.broadcasted_iota(jnp.int32, (8, _NR), 0)
        rcol = lax.broadcasted_iota(jnp.int32, (8, _NR), 1)
        sel = ((srow >> rcol) & 1).astype(jnp.float32)
        u = jnp.maximum(_dot(sel, m3) + base1, 0.0)
        zscr[...] = _dot(u, root2[...]) + bias2[...]
        tscr[...] = jnp.concatenate([_dot(u, w2[r]) for r in range(_NR)],
                                    axis=0)
        accg[...] = jnp.zeros_like(accg)
        accc[...] = jnp.zeros_like(accc)

    x = hp0[...] + hp1[...]
    degs = [jnp.sum(x[r * 8:(r + 1) * 8, :], axis=0, keepdims=True)
            for r in range(_NR)]
    xn = jnp.concatenate(
        [x[r * 8:(r + 1) * 8, :] / jnp.maximum(degs[r], 1.0)
         for r in range(_NR)], axis=0)
    state = (jnp.where(degs[0] > 0.0, 1, 0)
             + jnp.where(degs[1] > 0.0, 2, 0)
             + jnp.where(degs[2] > 0.0, 4, 0))
    o8 = (lax.broadcasted_iota(jnp.int32, (8, _NB), 0) == state
          ).astype(jnp.float32)
    h2 = jnp.maximum(_dotT(o8, zscr[...]) + _dotT(xn, tscr[...]), 0.0)

    bt = batch3[0, :, :]
    og = (lax.broadcasted_iota(jnp.int32, (_NG, _NB), 0) == bt
          ).astype(jnp.float32)
    accg[...] += _dot(og, h2)
    accc[...] += jnp.sum(og, axis=1, keepdims=True)

    @pl.when(i == _NBLK - 1)
    def _epilogue():
        gemb = accg[...] / jnp.maximum(accc[...], 1.0)
        out_ref[...] = _dot(gemb, lwp[...]) + lbp[...]


def kernel(x, edge_index, edge_type, batch, embed_table, w1, root1, bias1,
           w2, root2, bias2, lin_w, lin_b):
    del x
    src = edge_index[0]
    dst = edge_index[1]
    epad = _EPAD - _E
    src_p = jnp.concatenate([src, jnp.zeros((epad,), jnp.int32)])
    dst_p = jnp.concatenate([dst, jnp.zeros((epad,), jnp.int32)])
    ty_p = jnp.concatenate([edge_type, jnp.zeros((epad,), jnp.int32)])
    batch_p = jnp.concatenate(
        [batch.astype(jnp.int32),
         jnp.full((_NPAD - _N,), _NG, jnp.int32)]).reshape(_NBLK, 1, _NB)

    mesh = plsc.VectorSubcoreMesh(core_axis_name="c", subcore_axis_name="s")

    sc_a = functools.partial(
        pl.kernel, mesh=mesh,
        out_type=jax.ShapeDtypeStruct((2 * 3 * _NPAD,), jnp.float32),
        scratch_types=[
            pltpu.VMEM((_CH,), jnp.int32),
            pltpu.VMEM((_CH,), jnp.int32),
            pltpu.VMEM((_CH,), jnp.int32),
            pltpu.VMEM((_CH,), jnp.float32),
            pltpu.VMEM((_SL,), jnp.float32),
            pltpu.VMEM_SHARED((3 * _NPAD,), jnp.float32),
            pltpu.SemaphoreType.DMA,
        ])(_sc_a_body)
    cnt3 = sc_a(dst_p, ty_p)

    sc_b = functools.partial(
        pl.kernel, mesh=mesh,
        out_type=jax.ShapeDtypeStruct((2 * 24 * _NPAD,), jnp.float32),
        scratch_types=[
            pltpu.VMEM((_CH,), jnp.int32),
            pltpu.VMEM((_CH,), jnp.int32),
            pltpu.VMEM((_CH,), jnp.int32),
            pltpu.VMEM((_CH,), jnp.int32),
            pltpu.VMEM((_CH,), jnp.float32),
            pltpu.VMEM((_SL,), jnp.float32),
            pltpu.VMEM((_SL,), jnp.float32),
            pltpu.VMEM((_SL,), jnp.int32),
            pltpu.VMEM((_SL,), jnp.float32),
            pltpu.VMEM((_NPAD,), jnp.int32),
            pltpu.VMEM_SHARED((_NPAD,), jnp.int32),
            pltpu.VMEM_SHARED((24 * _NPAD,), jnp.float32),
            pltpu.SemaphoreType.DMA,
        ])(_sc_b_body)
    hist = sc_b(src_p, dst_p, ty_p, cnt3)

    hp = hist.reshape(2, 24, _NPAD)
    lwp = jnp.pad(lin_w, ((0, 0), (0, 128 - lin_w.shape[1])))
    lbp = jnp.pad(lin_b, (0, 128 - lin_b.shape[0])).reshape(1, 128)

    out = pl.pallas_call(
        _tc_body,
        grid=(_NBLK,),
        in_specs=[
            pl.BlockSpec((24, _NB), lambda i: (0, i)),
            pl.BlockSpec((24, _NB), lambda i: (0, i)),
            pl.BlockSpec((1, 1, _NB), lambda i: (i, 0, 0)),
            pl.BlockSpec((1, _D), lambda i: (0, 0)),
            pl.BlockSpec((_NR, _D, _H), lambda i: (0, 0, 0)),
            pl.BlockSpec((_D, _H), lambda i: (0, 0)),
            pl.BlockSpec((1, _H), lambda i: (0, 0)),
            pl.BlockSpec((_NR, _H, _H), lambda i: (0, 0, 0)),
            pl.BlockSpec((_H, _H), lambda i: (0, 0)),
            pl.BlockSpec((1, _H), lambda i: (0, 0)),
            pl.BlockSpec((_H, 128), lambda i: (0, 0)),
            pl.BlockSpec((1, 128), lambda i: (0, 0)),
        ],
        out_specs=pl.BlockSpec((_NG, 128), lambda i: (0, 0)),
        out_shape=jax.ShapeDtypeStruct((_NG, 128), jnp.float32),
        scratch_shapes=[
            pltpu.VMEM((24, _H), jnp.float32),
            pltpu.VMEM((8, _H), jnp.float32),
            pltpu.VMEM((_NG, _H), jnp.float32),
            pltpu.VMEM((_NG, 1), jnp.float32),
        ],
        compiler_params=pltpu.CompilerParams(
            dimension_semantics=("arbitrary",)),
    )(hp[0], hp[1], batch_p, embed_table, w1, root1,
      bias1.reshape(1, _H), w2, root2, bias2.reshape(1, _H), lwp, lbp)

    return out[:, :lin_w.shape[1]]


# speedup vs baseline: 57.9065x; 57.9065x over previous
import functools

import jax
import jax.numpy as jnp
from jax import lax
from jax.experimental import pallas as pl
from jax.experimental.pallas import tpu as pltpu
from jax.experimental.pallas import tpu_sc as plsc

_N = 50000
_E = 800000
_NR = 3
_D = 64
_H = 64
_NG = 128
_LANES = 16

_NPAD = 50176
_SL = _NPAD // 16

_CH = 2560
_KROWS = _CH // 128
_NSUB = 10
_PERW = _CH * _NSUB
_NW = 32
_EPAD = _PERW * _NW

_NBLK = 4
_NB = _NPAD // _NBLK


def _wid(c, s):
    return c * 16 + s


def _fill_zero(ref, nwords):
    def body(i, _):
        ref[pl.ds(i * _LANES, _LANES)] = jnp.zeros((_LANES,), jnp.float32)
        return 0
    lax.fori_loop(0, nwords // _LANES, body, 0)


def _sc_a_body(dst_hbm, ty_hbm, out_hbm, dstb, tyb, idxb, valb, idxb2, valb2,
               zb, spm, sem):
    c = lax.axis_index("c")
    s = lax.axis_index("s")
    wid = _wid(c, s)

    _fill_zero(zb, _SL)
    for k in range(3):
        pltpu.sync_copy(zb, spm.at[pl.ds(s * 3 * _SL + k * _SL, _SL)])
    plsc.subcore_barrier()

    ebase0 = pl.multiple_of(wid * _PERW, 128)

    def compute_chunk(j, idxp, valp):
        ebase = pl.multiple_of(ebase0 + j * _CH, 128)
        pltpu.sync_copy(dst_hbm.at[pl.ds(ebase, _CH)], dstb)
        pltpu.sync_copy(ty_hbm.at[pl.ds(ebase, _CH)], tyb)

        def row(k, _):
            for v in range(128 // _LANES):
                off = k * 128 + v * _LANES
                d = dstb[pl.ds(off, _LANES)]
                t = tyb[pl.ds(off, _LANES)]
                idxp[k, pl.ds(v * _LANES, _LANES)] = t * _NPAD + d
                gi = ebase + off + lax.iota(jnp.int32, _LANES)
                valp[k, pl.ds(v * _LANES, _LANES)] = jnp.where(gi < _E, 1.0, 0.0)
            return 0

        lax.fori_loop(0, _KROWS, row, 0)

    bufs = ((idxb, valb), (idxb2, valb2))
    for j in range(_NSUB):
        idxp, valp = bufs[j % 2]
        compute_chunk(j, idxp, valp)
        if j >= 1:
            idxq, valq = bufs[(j - 1) % 2]
            for k in range(_KROWS):
                pltpu.make_async_copy(valq.at[k], spm.at[idxq.at[k]], sem).wait()
        for k in range(_KROWS):
            pltpu.make_async_copy(valp.at[k], spm.at[idxp.at[k]], sem).start()
    idxp, valp = bufs[(_NSUB - 1) % 2]
    for k in range(_KROWS):
        pltpu.make_async_copy(valp.at[k], spm.at[idxp.at[k]], sem).wait()

    plsc.subcore_barrier()
    for k in range(3):
        pltpu.sync_copy(spm.at[pl.ds(s * 3 * _SL + k * _SL, _SL)], zb)
        pltpu.sync_copy(
            zb, out_hbm.at[pl.ds(c * 3 * _NPAD + s * 3 * _SL + k * _SL, _SL)])


_SLP = _SL // 8
_NPACK = _NPAD // 8


def _sc_b_body(src_hbm, dst_hbm, ty_hbm, cnt3_hbm, out_hbm,
               srcb, dstb, tyb, idxb, valb, idxb2, valb2, pa, pb, stb, packb,
               statebp, zb, spm_state, spm_hist, sem):
    c = lax.axis_index("c")
    s = lax.axis_index("s")
    wid = _wid(c, s)

    _fill_zero(zb, _SL)
    for k in range(24):
        pltpu.sync_copy(zb, spm_hist.at[pl.ds(s * 24 * _SL + k * _SL, _SL)])

    def zvec(i, _):
        stb[pl.ds(_SL + i * _LANES, _LANES)] = jnp.zeros((_LANES,), jnp.int32)
        return 0

    lax.fori_loop(0, 4, zvec, 0)
    for r in range(3):
        pltpu.sync_copy(cnt3_hbm.at[pl.ds(r * _NPAD + s * _SL, _SL)], pa)
        pltpu.sync_copy(cnt3_hbm.at[pl.ds(3 * _NPAD + r * _NPAD + s * _SL, _SL)], pb)

        def vec(i, _):
            off = i * _LANES
            deg = pa[pl.ds(off, _LANES)] + pb[pl.ds(off, _LANES)]
            bit = jnp.where(deg > 0.0, jnp.int32(1 << r), jnp.int32(0))
            if r == 0:
                stb[pl.ds(off, _LANES)] = bit
            else:
                stb[pl.ds(off, _LANES)] = stb[pl.ds(off, _LANES)] + bit
            return 0

        lax.fori_loop(0, _SL // _LANES, vec, 0)

    def pvec(j, _):
        base = j * 128
        acc = jnp.zeros((_LANES,), jnp.int32)
        for b in range(8):
            idxs = base + lax.iota(jnp.int32, _LANES) * 8 + b
            acc = acc | (plsc.load_gather(stb, [idxs]) << (4 * b))
        packb[pl.ds(j * _LANES, _LANES)] = acc
        return 0

    lax.fori_loop(0, (_SLP + _LANES - 1) // _LANES, pvec, 0)
    pltpu.sync_copy(packb.at[pl.ds(0, _SLP)], spm_state.at[pl.ds(s * _SLP, _SLP)])
    plsc.subcore_barrier()

    pltpu.sync_copy(spm_state, statebp)

    ebase0 = pl.multiple_of(wid * _PERW, 128)

    def compute_chunk(j, idxp, valp):
        ebase = pl.multiple_of(ebase0 + j * _CH, 128)
        pltpu.sync_copy(src_hbm.at[pl.ds(ebase, _CH)], srcb)
        pltpu.sync_copy(dst_hbm.at[pl.ds(ebase, _CH)], dstb)
        pltpu.sync_copy(ty_hbm.at[pl.ds(ebase, _CH)], tyb)

        def row(k, _):
            for v in range(128 // _LANES):
                off = k * 128 + v * _LANES
                sv = srcb[pl.ds(off, _LANES)]
                w = plsc.load_gather(statebp, [sv >> 3])
                st = (w >> ((sv & 7) << 2)) & 7
                t = tyb[pl.ds(off, _LANES)]
                d = dstb[pl.ds(off, _LANES)]
                idxp[k, pl.ds(v * _LANES, _LANES)] = (t * 8 + st) * _NPAD + d
                gi = ebase + off + lax.iota(jnp.int32, _LANES)
                valp[k, pl.ds(v * _LANES, _LANES)] = jnp.where(gi < _E, 1.0, 0.0)
            return 0

        lax.fori_loop(0, _KROWS, row, 0)

    bufs = ((idxb, valb), (idxb2, valb2))
    for j in range(_NSUB):
        idxp, valp = bufs[j % 2]
        compute_chunk(j, idxp, valp)
        if j >= 1:
            idxq, valq = bufs[(j - 1) % 2]
            for k in range(_KROWS):
                pltpu.make_async_copy(
                    valq.at[k], spm_hist.at[idxq.at[k]], sem).wait()
        for k in range(_KROWS):
            pltpu.make_async_copy(
                valp.at[k], spm_hist.at[idxp.at[k]], sem).start()
    idxp, valp = bufs[(_NSUB - 1) % 2]
    for k in range(_KROWS):
        pltpu.make_async_copy(
            valp.at[k], spm_hist.at[idxp.at[k]], sem).wait()

    plsc.subcore_barrier()

    for k in range(24):
        pltpu.sync_copy(spm_hist.at[pl.ds(s * 24 * _SL + k * _SL, _SL)], zb)
        pltpu.sync_copy(
            zb, out_hbm.at[pl.ds(c * 24 * _NPAD + s * 24 * _SL + k * _SL, _SL)])


def _dotT(a, b):
    return lax.dot_general(a, b, (((0,), (0,)), ((), ())),
                           precision=lax.Precision.HIGHEST,
                           preferred_element_type=jnp.float32)


def _dot(a, b):
    return lax.dot_general(a, b, (((1,), (0,)), ((), ())),
                           precision=lax.Precision.HIGHEST,
                           preferred_element_type=jnp.float32)


def _dot_ref(a, b):
    return lax.dot_general(a, b, (((1,), (0,)), ((), ())),
                           precision=lax.Precision.DEFAULT,
                           preferred_element_type=jnp.float32)


def _dot_pool(a, b):
    return _dot(a, b)


def _tc_body(hp0, hp1, batch3, emb, w1, root1, bias1, w2, root2, bias2,
             lwp, lbp, out_ref, tscr, zscr, accg, accc):
    i = pl.program_id(0)

    @pl.when(i == 0)
    def _prologue():
        v = emb[...]
        base1 = _dot_ref(v, root1[...]) + bias1[...]
        m3 = jnp.concatenate([_dot_ref(v, w1[r]) for r in range(_NR)], axis=0)
        srow = lax.broadcasted_iota(jnp.int32, (8, _NR), 0)
        rcol = lax.broadcasted_iota(jnp.int32, (8, _NR), 1)
        sel = ((srow >> rcol) & 1).astype(jnp.float32)
        u = jnp.maximum(_dot(sel, m3) + base1, 0.0)
        zscr[...] = _dot_ref(u, root2[...]) + bias2[...]
        tscr[...] = jnp.concatenate([_dot_ref(u, w2[r]) for r in range(_NR)],
                                    axis=0)
        accg[...] = jnp.zeros_like(accg)
        accc[...] = jnp.zeros_like(accc)

    x = hp0[...] + hp1[...]
    degs = [jnp.sum(x[r * 8:(r + 1) * 8, :], axis=0, keepdims=True)
            for r in range(_NR)]
    xn = jnp.concatenate(
        [x[r * 8:(r + 1) * 8, :] / jnp.maximum(degs[r], 1.0)
         for r in range(_NR)], axis=0)
    state = (jnp.where(degs[0] > 0.0, 1, 0)
             + jnp.where(degs[1] > 0.0, 2, 0)
             + jnp.where(degs[2] > 0.0, 4, 0))
    o8 = (lax.broadcasted_iota(jnp.int32, (8, _NB), 0) == state
          ).astype(jnp.float32)
    h2 = jnp.maximum(_dotT(o8, zscr[...]) + _dotT(xn, tscr[...]), 0.0)

    bt = batch3[0, :, :]
    og = (lax.broadcasted_iota(jnp.int32, (_NG, _NB), 0) == bt
          ).astype(jnp.float32)
    accg[...] += _dot_pool(og, h2)
    accc[...] += jnp.sum(og, axis=1, keepdims=True)

    @pl.when(i == _NBLK - 1)
    def _epilogue():
        gemb = accg[...] / jnp.maximum(accc[...], 1.0)
        out_ref[...] = _dot_ref(gemb, lwp[...]) + lbp[...]


def kernel(x, edge_index, edge_type, batch, embed_table, w1, root1, bias1,
           w2, root2, bias2, lin_w, lin_b):
    del x
    src = edge_index[0]
    dst = edge_index[1]
    epad = _EPAD - _E
    src_p = jnp.concatenate([src, jnp.zeros((epad,), jnp.int32)])
    dst_p = jnp.concatenate([dst, jnp.zeros((epad,), jnp.int32)])
    ty_p = jnp.concatenate([edge_type, jnp.zeros((epad,), jnp.int32)])
    batch_p = jnp.concatenate(
        [batch.astype(jnp.int32),
         jnp.full((_NPAD - _N,), _NG, jnp.int32)]).reshape(_NBLK, 1, _NB)

    mesh = plsc.VectorSubcoreMesh(core_axis_name="c", subcore_axis_name="s")

    sc_a = functools.partial(
        pl.kernel, mesh=mesh,
        out_type=jax.ShapeDtypeStruct((2 * 3 * _NPAD,), jnp.float32),
        scratch_types=[
            pltpu.VMEM((_CH,), jnp.int32),
            pltpu.VMEM((_CH,), jnp.int32),
            pltpu.VMEM((_KROWS, 128), jnp.int32),
            pltpu.VMEM((_KROWS, 128), jnp.float32),
            pltpu.VMEM((_KROWS, 128), jnp.int32),
            pltpu.VMEM((_KROWS, 128), jnp.float32),
            pltpu.VMEM((_SL,), jnp.float32),
            pltpu.VMEM_SHARED((3 * _NPAD,), jnp.float32),
            pltpu.SemaphoreType.DMA,
        ],
        compiler_params=pltpu.CompilerParams(
            use_tc_tiling_on_sc=False, needs_layout_passes=False))(_sc_a_body)
    cnt3 = sc_a(dst_p, ty_p)

    sc_b = functools.partial(
        pl.kernel, mesh=mesh,
        out_type=jax.ShapeDtypeStruct((2 * 24 * _NPAD,), jnp.float32),
        scratch_types=[
            pltpu.VMEM((_CH,), jnp.int32),
            pltpu.VMEM((_CH,), jnp.int32),
            pltpu.VMEM((_CH,), jnp.int32),
            pltpu.VMEM((_KROWS, 128), jnp.int32),
            pltpu.VMEM((_KROWS, 128), jnp.float32),
            pltpu.VMEM((_KROWS, 128), jnp.int32),
            pltpu.VMEM((_KROWS, 128), jnp.float32),
            pltpu.VMEM((_SL,), jnp.float32),
            pltpu.VMEM((_SL,), jnp.float32),
            pltpu.VMEM((_SL + 64,), jnp.int32),
            pltpu.VMEM((_SLP + 8,), jnp.int32),
            pltpu.VMEM((_NPACK,), jnp.int32),
            pltpu.VMEM((_SL,), jnp.float32),
            pltpu.VMEM_SHARED((_NPACK,), jnp.int32),
            pltpu.VMEM_SHARED((24 * _NPAD,), jnp.float32),
            pltpu.SemaphoreType.DMA,
        ],
        compiler_params=pltpu.CompilerParams(
            use_tc_tiling_on_sc=False, needs_layout_passes=False))(_sc_b_body)
    hist = sc_b(src_p, dst_p, ty_p, cnt3)

    hp = hist.reshape(2, 24, _NPAD)
    lwp = jnp.pad(lin_w, ((0, 0), (0, 128 - lin_w.shape[1])))
    lbp = jnp.pad(lin_b, (0, 128 - lin_b.shape[0])).reshape(1, 128)

    out = pl.pallas_call(
        _tc_body,
        grid=(_NBLK,),
        in_specs=[
            pl.BlockSpec((24, _NB), lambda i: (0, i)),
            pl.BlockSpec((24, _NB), lambda i: (0, i)),
            pl.BlockSpec((1, 1, _NB), lambda i: (i, 0, 0)),
            pl.BlockSpec((1, _D), lambda i: (0, 0)),
            pl.BlockSpec((_NR, _D, _H), lambda i: (0, 0, 0)),
            pl.BlockSpec((_D, _H), lambda i: (0, 0)),
            pl.BlockSpec((1, _H), lambda i: (0, 0)),
            pl.BlockSpec((_NR, _H, _H), lambda i: (0, 0, 0)),
            pl.BlockSpec((_H, _H), lambda i: (0, 0)),
            pl.BlockSpec((1, _H), lambda i: (0, 0)),
            pl.BlockSpec((_H, 128), lambda i: (0, 0)),
            pl.BlockSpec((1, 128), lambda i: (0, 0)),
        ],
        out_specs=pl.BlockSpec((_NG, 128), lambda i: (0, 0)),
        out_shape=jax.ShapeDtypeStruct((_NG, 128), jnp.float32),
        scratch_shapes=[
            pltpu.VMEM((24, _H), jnp.float32),
            pltpu.VMEM((8, _H), jnp.float32),
            pltpu.VMEM((_NG, _H), jnp.float32),
            pltpu.VMEM((_NG, 1), jnp.float32),
        ],
        compiler_params=pltpu.CompilerParams(
            dimension_semantics=("arbitrary",)),
    )(hp[0], hp[1], batch_p, embed_table, w1, root1,
      bias1.reshape(1, _H), w2, root2, bias2.reshape(1, _H), lwp, lbp)

    return out[:, :lin_w.shape[1]]
